# baseline (speedup 1.0000x reference)
import jax
import jax.numpy as jnp
from jax import lax
from jax.experimental import pallas as pl
from jax.experimental.pallas import tpu as pltpu
from jax.experimental.pallas import tpu_sc as plsc

N = 10000
E = 320000
D = 128
C = 40

NC = 2
NS = 16
NH = N // NC
NTRASH = 8
ACC_R = NH + NTRASH
EPS = E // NS
SLAB = EPS + 96
K = 32
NCH = EPS // K
RZ = 312
TAIL = NH - NS * RZ


def _sc_mesh():
  return plsc.VectorSubcoreMesh(
      core_axis_name="c", subcore_axis_name="s",
      num_cores=NC, num_subcores=NS)


def _make_seg_sum():
  scratch = [
      pltpu.VMEM((SLAB,), jnp.int32),
      pltpu.VMEM((SLAB,), jnp.int32),
      pltpu.VMEM((2, K), jnp.int32),
      pltpu.VMEM((2, K, D), jnp.float32),
      pltpu.VMEM((RZ, D), jnp.float32),
      pltpu.VMEM_SHARED((ACC_R, D), jnp.float32),
      pltpu.SemaphoreType.DMA,
      pltpu.SemaphoreType.DMA,
  ]

  def body(h_hbm, src_hbm, dst_hbm, out_hbm,
           src_all, dst_all, idx2, rows2, zbuf, acc, gsem, ssem):
    c = lax.axis_index("c")
    s = lax.axis_index("s")
    z16 = jnp.zeros((16,), jnp.float32)
    lo = c * NH

    def zrow(i, carry):
      for j in range(D // 16):
        zbuf[i, pl.ds(j * 16, 16)] = z16
      return carry
    lax.fori_loop(0, RZ, zrow, 0)
    pltpu.sync_copy(zbuf, acc.at[pl.ds(s * RZ, RZ)])

    @pl.when(s == 0)
    def _zero_tail():
      pltpu.sync_copy(zbuf.at[pl.ds(0, TAIL + NTRASH)],
                      acc.at[pl.ds(NS * RZ, TAIL + NTRASH)])

    base_al = (s * EPS) // 128 * 128
    off = s * EPS - base_al
    pltpu.sync_copy(src_hbm.at[pl.ds(base_al, SLAB)], src_all)
    pltpu.sync_copy(dst_hbm.at[pl.ds(base_al, SLAB)], dst_all)
    plsc.subcore_barrier()

    def gidx(t):
      return src_all.at[pl.ds(off + t * K, K)]

    def remap(t, par):
      for j in range(K // 16):
        dv = dst_all[pl.ds(off + t * K + j * 16, 16)]
        v = dv - lo
        ok = (v >= 0) & (v < NH)
        idx2[par, pl.ds(j * 16, 16)] = jnp.where(ok, v,
                                                 NH + (dv & (NTRASH - 1)))

    def sidx(t):
      return idx2.at[t & 1]

    def step(t, carry):
      @pl.when(t >= 2)
      def _wait_scatter():
        pltpu.make_async_copy(rows2.at[t & 1], acc.at[sidx(t - 2)],
                              ssem).wait()

      @pl.when(t < NCH)
      def _fire_gather():
        pltpu.async_copy(h_hbm.at[gidx(t)], rows2.at[t & 1], gsem)

      @pl.when((t >= 1) & (t <= NCH))
      def _drain_fire():
        tp = t - 1
        remap(tp, tp & 1)
        pltpu.make_async_copy(h_hbm.at[gidx(tp)], rows2.at[tp & 1],
                              gsem).wait()
        pltpu.async_copy(rows2.at[tp & 1], acc.at[sidx(tp)], ssem,
                         add=True)
      return carry
    lax.fori_loop(0, NCH + 2, step, 0)

    plsc.subcore_barrier()
    sl = pl.ds(s * RZ, RZ)
    pltpu.sync_copy(acc.at[sl], out_hbm.at[c, sl])

    @pl.when(s == 0)
    def _out_tail():
      tl = pl.ds(NS * RZ, TAIL)
      pltpu.sync_copy(acc.at[tl], out_hbm.at[c, tl])

  return pl.kernel(
      body,
      out_type=jax.ShapeDtypeStruct((NC, NH, D), jnp.float32),
      mesh=_sc_mesh(), scratch_types=tuple(scratch))


def _make_seg_cnt():
  scratch = [
      pltpu.VMEM((SLAB,), jnp.int32),
      pltpu.VMEM((K,), jnp.int32),
      pltpu.VMEM((K, D), jnp.float32),
      pltpu.VMEM((RZ, D), jnp.float32),
      pltpu.VMEM_SHARED((ACC_R, D), jnp.float32),
  ]

  def body(dst_hbm, cnt_hbm, dst_all, idx_v, ones_v, zbuf, cacc):
    c = lax.axis_index("c")
    s = lax.axis_index("s")
    z16 = jnp.zeros((16,), jnp.float32)
    one16 = jnp.ones((16,), jnp.float32)
    lo = c * NH

    def orow(i, carry):
      for j in range(D // 16):
        ones_v[i, pl.ds(j * 16, 16)] = one16
      return carry
    lax.fori_loop(0, K, orow, 0)

    def zrow(i, carry):
      for j in range(D // 16):
        zbuf[i, pl.ds(j * 16, 16)] = z16
      return carry
    lax.fori_loop(0, RZ, zrow, 0)
    pltpu.sync_copy(zbuf, cacc.at[pl.ds(s * RZ, RZ)])

    @pl.when(s == 0)
    def _zero_tail():
      pltpu.sync_copy(zbuf.at[pl.ds(0, TAIL + NTRASH)],
                      cacc.at[pl.ds(NS * RZ, TAIL + NTRASH)])

    base_al = (s * EPS) // 128 * 128
    off = s * EPS - base_al
    pltpu.sync_copy(dst_hbm.at[pl.ds(base_al, SLAB)], dst_all)
    plsc.subcore_barrier()

    def step(t, carry):
      for j in range(K // 16):
        dv = dst_all[pl.ds(off + t * K + j * 16, 16)]
        v = dv - lo
        ok = (v >= 0) & (v < NH)
        idx_v[pl.ds(j * 16, 16)] = jnp.where(ok, v, NH + (dv & (NTRASH - 1)))
      pltpu.sync_copy(ones_v, cacc.at[idx_v], add=True)
      return carry
    lax.fori_loop(0, NCH, step, 0)

    plsc.subcore_barrier()
    sl = pl.ds(s * RZ, RZ)
    pltpu.sync_copy(cacc.at[sl], cnt_hbm.at[c, sl])

    @pl.when(s == 0)
    def _cnt_tail():
      tl = pl.ds(NS * RZ, TAIL)
      pltpu.sync_copy(cacc.at[tl], cnt_hbm.at[c, tl])

  return pl.kernel(
      body,
      out_type=jax.ShapeDtypeStruct((NC, NH, D), jnp.float32),
      mesh=_sc_mesh(), scratch_types=tuple(scratch))


_seg_sum = _make_seg_sum()
_seg_cnt = _make_seg_cnt()


def _dotT(a, w):
  return lax.dot_general(a, w, (((1,), (1,)), ((), ())),
                         preferred_element_type=jnp.float32)


def _bn_relu(y, g, be):
  y = jnp.maximum(y, 0.0)
  mu = jnp.mean(y, axis=0, keepdims=True)
  var = jnp.mean((y - mu) ** 2, axis=0, keepdims=True)
  return (y - mu) * lax.rsqrt(var + 1e-5) * g + be


def _full(sp_ref):
  return jnp.concatenate([sp_ref[0], sp_ref[1]], axis=0)


def _tc_layer0(sp, cntp, x, wl, wr, b, g, be):
  def tc_body(sp_ref, cnt_ref, x_ref, wl_ref, wr_ref, b_ref, g_ref, be_ref,
              h_out, ic_out):
    cnt = _full(cnt_ref)[:, 0:1]
    ic = 1.0 / jnp.maximum(cnt, 1.0)
    mean = _full(sp_ref) * ic
    y = _dotT(mean, wl_ref[...]) + _dotT(x_ref[...], wr_ref[...]) + b_ref[...]
    h_out[...] = _bn_relu(y, g_ref[...], be_ref[...])
    ic_out[...] = ic
  return pl.pallas_call(
      tc_body,
      out_shape=(jax.ShapeDtypeStruct((N, D), jnp.float32),
                 jax.ShapeDtypeStruct((N, 1), jnp.float32)),
  )(sp, cntp, x, wl, wr, b, g, be)


def _tc_mid(sp, h, ic, wl, wr, b, g, be):
  def tc_body(sp_ref, h_ref, ic_ref, wl_ref, wr_ref, b_ref, g_ref, be_ref,
              h_out):
    mean = _full(sp_ref) * ic_ref[...]
    y = _dotT(mean, wl_ref[...]) + _dotT(h_ref[...], wr_ref[...]) + b_ref[...]
    h_out[...] = _bn_relu(y, g_ref[...], be_ref[...])
  return pl.pallas_call(
      tc_body,
      out_shape=jax.ShapeDtypeStruct((N, D), jnp.float32),
  )(sp, h, ic, wl, wr, b, g, be)


def _tc_final(sp, h, ic, wlf, wrf, bf):
  def tc_body(sp_ref, h_ref, ic_ref, wlf_ref, wrf_ref, bf_ref, o_ref):
    mean = _full(sp_ref) * ic_ref[...]
    y = _dotT(mean, wlf_ref[...]) + _dotT(h_ref[...], wrf_ref[...]) \
        + bf_ref[...]
    mx = jnp.max(y, axis=1, keepdims=True)
    z = y - mx
    lse = jnp.log(jnp.sum(jnp.exp(z), axis=1, keepdims=True))
    o_ref[...] = z - lse
  return pl.pallas_call(
      tc_body,
      out_shape=jax.ShapeDtypeStruct((N, C), jnp.float32),
  )(sp, h, ic, wlf, wrf, bf)


def kernel(x, edge_index, Wl0, Wr0, b0, g0, be0, Wl1, Wr1, b1, g1, be1,
           Wl2, Wr2, b2, g2, be2, Wl3, Wr3, b3, g3, be3, Wlf, Wrf, bf):
  r = lambda v: v.reshape(1, -1)
  src, dst = edge_index[0], edge_index[1]

  cnt = _seg_cnt(dst)
  s0 = _seg_sum(x, src, dst)
  h1, ic = _tc_layer0(s0, cnt, x, Wl0, Wr0, r(b0), r(g0), r(be0))
  s1 = _seg_sum(h1, src, dst)
  h2 = _tc_mid(s1, h1, ic, Wl1, Wr1, r(b1), r(g1), r(be1))
  s2 = _seg_sum(h2, src, dst)
  h3 = _tc_mid(s2, h2, ic, Wl2, Wr2, r(b2), r(g2), r(be2))
  s3 = _seg_sum(h3, src, dst)
  h4 = _tc_mid(s3, h3, ic, Wl3, Wr3, r(b3), r(g3), r(be3))
  s4 = _seg_sum(h4, src, dst)
  return _tc_final(s4, h4, ic, Wlf, Wrf, r(bf))


# speedup vs baseline: 1.0195x; 1.0195x over previous
import jax
import jax.numpy as jnp
from jax import lax
from jax.experimental import pallas as pl
from jax.experimental.pallas import tpu as pltpu
from jax.experimental.pallas import tpu_sc as plsc

N = 10000
E = 320000
D = 128
C = 40

NC = 2
NS = 16
NH = N // NC
NTRASH = 8
ACC_R = NH + NTRASH
EPS = E // NS
SLAB = EPS + 96
K = 32
NCH = EPS // K
RZ = 312
TAIL = NH - NS * RZ


def _sc_mesh():
  return plsc.VectorSubcoreMesh(
      core_axis_name="c", subcore_axis_name="s",
      num_cores=NC, num_subcores=NS)


def _make_seg_sum():
  scratch = [
      pltpu.VMEM((SLAB,), jnp.int32),
      pltpu.VMEM((SLAB,), jnp.int32),
      pltpu.VMEM((2, K), jnp.int32),
      pltpu.VMEM((2, K, D), jnp.float32),
      pltpu.VMEM((RZ, D), jnp.float32),
      pltpu.VMEM_SHARED((ACC_R, D), jnp.float32),
      pltpu.SemaphoreType.DMA,
      pltpu.SemaphoreType.DMA,
  ]

  def body(h_hbm, src_hbm, dst_hbm, out_hbm,
           src_all, dst_all, idx2, rows2, zbuf, acc, gsem, ssem):
    c = lax.axis_index("c")
    s = lax.axis_index("s")
    z16 = jnp.zeros((16,), jnp.float32)
    lo = c * NH

    def zrow(i, carry):
      for j in range(D // 16):
        zbuf[i, pl.ds(j * 16, 16)] = z16
      return carry
    lax.fori_loop(0, RZ, zrow, 0)
    pltpu.sync_copy(zbuf, acc.at[pl.ds(s * RZ, RZ)])

    @pl.when(s == 0)
    def _zero_tail():
      pltpu.sync_copy(zbuf.at[pl.ds(0, TAIL + NTRASH)],
                      acc.at[pl.ds(NS * RZ, TAIL + NTRASH)])

    base_al = (s * EPS) // 128 * 128
    off = s * EPS - base_al
    pltpu.sync_copy(src_hbm.at[pl.ds(base_al, SLAB)], src_all)
    pltpu.sync_copy(dst_hbm.at[pl.ds(base_al, SLAB)], dst_all)
    plsc.subcore_barrier()

    def gidx(t):
      return src_all.at[pl.ds(off + t * K, K)]

    def remap(t, par):
      for j in range(K // 16):
        dv = dst_all[pl.ds(off + t * K + j * 16, 16)]
        v = dv - lo
        ok = (v >= 0) & (v < NH)
        idx2[par, pl.ds(j * 16, 16)] = jnp.where(ok, v,
                                                 NH + (dv & (NTRASH - 1)))

    def sidx(t):
      return idx2.at[t & 1]

    def step(t, carry):
      @pl.when(t >= 2)
      def _wait_scatter():
        pltpu.make_async_copy(rows2.at[t & 1], acc.at[sidx(t - 2)],
                              ssem).wait()

      @pl.when(t < NCH)
      def _fire_gather():
        pltpu.async_copy(h_hbm.at[gidx(t)], rows2.at[t & 1], gsem)

      @pl.when((t >= 1) & (t <= NCH))
      def _drain_fire():
        tp = t - 1
        remap(tp, tp & 1)
        pltpu.make_async_copy(h_hbm.at[gidx(tp)], rows2.at[tp & 1],
                              gsem).wait()
        pltpu.async_copy(rows2.at[tp & 1], acc.at[sidx(tp)], ssem,
                         add=True)
      return carry
    lax.fori_loop(0, NCH + 2, step, 0)

    plsc.subcore_barrier()
    sl = pl.ds(s * RZ, RZ)
    pltpu.sync_copy(acc.at[sl], out_hbm.at[c, sl])

    @pl.when(s == 0)
    def _out_tail():
      tl = pl.ds(NS * RZ, TAIL)
      pltpu.sync_copy(acc.at[tl], out_hbm.at[c, tl])

  return pl.kernel(
      body,
      out_type=jax.ShapeDtypeStruct((NC, NH, D), jnp.float32),
      mesh=_sc_mesh(), scratch_types=tuple(scratch))


def _make_seg_cnt():
  scratch = [
      pltpu.VMEM((SLAB,), jnp.int32),
      pltpu.VMEM((2, K), jnp.int32),
      pltpu.VMEM((K, D), jnp.float32),
      pltpu.VMEM((RZ, D), jnp.float32),
      pltpu.VMEM_SHARED((ACC_R, D), jnp.float32),
      pltpu.SemaphoreType.DMA,
  ]

  def body(dst_hbm, cnt_hbm, dst_all, idx2, ones_v, zbuf, cacc, ssem):
    c = lax.axis_index("c")
    s = lax.axis_index("s")
    z16 = jnp.zeros((16,), jnp.float32)
    one16 = jnp.ones((16,), jnp.float32)
    lo = c * NH

    def orow(i, carry):
      for j in range(D // 16):
        ones_v[i, pl.ds(j * 16, 16)] = one16
      return carry
    lax.fori_loop(0, K, orow, 0)

    def zrow(i, carry):
      for j in range(D // 16):
        zbuf[i, pl.ds(j * 16, 16)] = z16
      return carry
    lax.fori_loop(0, RZ, zrow, 0)
    pltpu.sync_copy(zbuf, cacc.at[pl.ds(s * RZ, RZ)])

    @pl.when(s == 0)
    def _zero_tail():
      pltpu.sync_copy(zbuf.at[pl.ds(0, TAIL + NTRASH)],
                      cacc.at[pl.ds(NS * RZ, TAIL + NTRASH)])

    base_al = (s * EPS) // 128 * 128
    off = s * EPS - base_al
    pltpu.sync_copy(dst_hbm.at[pl.ds(base_al, SLAB)], dst_all)
    plsc.subcore_barrier()

    def step(t, carry):
      @pl.when(t >= 2)
      def _wait_scatter():
        pltpu.make_async_copy(ones_v, cacc.at[idx2.at[t & 1]], ssem).wait()

      @pl.when(t < NCH)
      def _fire():
        for j in range(K // 16):
          dv = dst_all[pl.ds(off + t * K + j * 16, 16)]
          v = dv - lo
          ok = (v >= 0) & (v < NH)
          idx2[t & 1, pl.ds(j * 16, 16)] = jnp.where(
              ok, v, NH + (dv & (NTRASH - 1)))
        pltpu.async_copy(ones_v, cacc.at[idx2.at[t & 1]], ssem, add=True)
      return carry
    lax.fori_loop(0, NCH + 2, step, 0)

    plsc.subcore_barrier()
    sl = pl.ds(s * RZ, RZ)
    pltpu.sync_copy(cacc.at[sl], cnt_hbm.at[c, sl])

    @pl.when(s == 0)
    def _cnt_tail():
      tl = pl.ds(NS * RZ, TAIL)
      pltpu.sync_copy(cacc.at[tl], cnt_hbm.at[c, tl])

  return pl.kernel(
      body,
      out_type=jax.ShapeDtypeStruct((NC, NH, D), jnp.float32),
      mesh=_sc_mesh(), scratch_types=tuple(scratch))


_seg_sum = _make_seg_sum()
_seg_cnt = _make_seg_cnt()


def _dotT(a, w):
  return lax.dot_general(a, w, (((1,), (1,)), ((), ())),
                         preferred_element_type=jnp.float32)


def _bn_relu(y, g, be):
  y = jnp.maximum(y, 0.0)
  mu = jnp.mean(y, axis=0, keepdims=True)
  var = jnp.mean((y - mu) ** 2, axis=0, keepdims=True)
  return (y - mu) * lax.rsqrt(var + 1e-5) * g + be


def _full(sp_ref):
  return jnp.concatenate([sp_ref[0], sp_ref[1]], axis=0)


def _tc_layer0(sp, cntp, x, wl, wr, b, g, be):
  def tc_body(sp_ref, cnt_ref, x_ref, wl_ref, wr_ref, b_ref, g_ref, be_ref,
              h_out, ic_out):
    cnt = _full(cnt_ref)[:, 0:1]
    ic = 1.0 / jnp.maximum(cnt, 1.0)
    mean = _full(sp_ref) * ic
    y = _dotT(mean, wl_ref[...]) + _dotT(x_ref[...], wr_ref[...]) + b_ref[...]
    h_out[...] = _bn_relu(y, g_ref[...], be_ref[...])
    ic_out[...] = ic
  return pl.pallas_call(
      tc_body,
      out_shape=(jax.ShapeDtypeStruct((N, D), jnp.float32),
                 jax.ShapeDtypeStruct((N, 1), jnp.float32)),
  )(sp, cntp, x, wl, wr, b, g, be)


def _tc_mid(sp, h, ic, wl, wr, b, g, be):
  def tc_body(sp_ref, h_ref, ic_ref, wl_ref, wr_ref, b_ref, g_ref, be_ref,
              h_out):
    mean = _full(sp_ref) * ic_ref[...]
    y = _dotT(mean, wl_ref[...]) + _dotT(h_ref[...], wr_ref[...]) + b_ref[...]
    h_out[...] = _bn_relu(y, g_ref[...], be_ref[...])
  return pl.pallas_call(
      tc_body,
      out_shape=jax.ShapeDtypeStruct((N, D), jnp.float32),
  )(sp, h, ic, wl, wr, b, g, be)


def _tc_final(sp, h, ic, wlf, wrf, bf):
  def tc_body(sp_ref, h_ref, ic_ref, wlf_ref, wrf_ref, bf_ref, o_ref):
    mean = _full(sp_ref) * ic_ref[...]
    y = _dotT(mean, wlf_ref[...]) + _dotT(h_ref[...], wrf_ref[...]) \
        + bf_ref[...]
    mx = jnp.max(y, axis=1, keepdims=True)
    z = y - mx
    lse = jnp.log(jnp.sum(jnp.exp(z), axis=1, keepdims=True))
    o_ref[...] = z - lse
  return pl.pallas_call(
      tc_body,
      out_shape=jax.ShapeDtypeStruct((N, C), jnp.float32),
  )(sp, h, ic, wlf, wrf, bf)


def kernel(x, edge_index, Wl0, Wr0, b0, g0, be0, Wl1, Wr1, b1, g1, be1,
           Wl2, Wr2, b2, g2, be2, Wl3, Wr3, b3, g3, be3, Wlf, Wrf, bf):
  r = lambda v: v.reshape(1, -1)
  src, dst = edge_index[0], edge_index[1]

  cnt = _seg_cnt(dst)
  s0 = _seg_sum(x, src, dst)
  h1, ic = _tc_layer0(s0, cnt, x, Wl0, Wr0, r(b0), r(g0), r(be0))
  s1 = _seg_sum(h1, src, dst)
  h2 = _tc_mid(s1, h1, ic, Wl1, Wr1, r(b1), r(g1), r(be1))
  s2 = _seg_sum(h2, src, dst)
  h3 = _tc_mid(s2, h2, ic, Wl2, Wr2, r(b2), r(g2), r(be2))
  s3 = _seg_sum(h3, src, dst)
  h4 = _tc_mid(s3, h3, ic, Wl3, Wr3, r(b3), r(g3), r(be3))
  s4 = _seg_sum(h4, src, dst)
  return _tc_final(s4, h4, ic, Wlf, Wrf, r(bf))
